# baseline (speedup 1.0000x reference)
import jax
import jax.numpy as jnp
from jax import lax
from jax.experimental import pallas as pl
from jax.experimental.pallas import tpu as pltpu
from jax.experimental.pallas import tpu_sc as plsc

VOCAB = 1000000
EMBED_DIM = 100
MAX_LEN = 200
BATCH = 4096

NUM_WORKERS = 32
ROWS_PER_WORKER = BATCH // NUM_WORKERS
HALF_L = MAX_LEN // 2
DPAD = 112
NCHUNK = DPAD // 16
_NACC = 4


def _sc_delta_kernel(idx_hbm, wd_hbm, table_hbm, d_hbm,
                     idx_v, wd_v, bufs, dbuf, sems):
    wid = lax.axis_index("s") * 2 + lax.axis_index("c")

    pltpu.sync_copy(idx_hbm.at[wid], idx_v)
    pltpu.sync_copy(wd_hbm, wd_v)

    def gather_start(slot, b):
        for j in range(2):
            pltpu.make_async_copy(
                table_hbm.at[idx_v.at[b, j]], bufs.at[slot, j], sems.at[slot]
            ).start()

    def gather_wait(slot, b):
        for j in range(2):
            pltpu.make_async_copy(
                table_hbm.at[idx_v.at[b, j]], bufs.at[slot, j], sems.at[slot]
            ).wait()

    def compute(slot, b):
        accs = [jnp.zeros((16,), jnp.float32) for _ in range(_NACC)]

        def lbody(l, accs):
            accs = list(accs)
            k = 0
            for j in range(2):
                for c in range(NCHUNK):
                    r = bufs[slot, j, l, pl.ds(c * 16, 16)]
                    w = wd_v[j * HALF_L + l, pl.ds(c * 16, 16)]
                    accs[k % _NACC] = accs[k % _NACC] + r * w
                    k += 1
            return tuple(accs)

        accs = lax.fori_loop(0, HALF_L, lbody, tuple(accs), unroll=2)
        dbuf[b] = accs[0] + accs[1] + accs[2] + accs[3]

    gather_start(0, 0)
    gather_start(1, 1)

    def group(g, carry):
        b0 = 2 * g
        gather_wait(0, b0)
        compute(0, b0)

        @pl.when(b0 + 2 < ROWS_PER_WORKER)
        def _():
            gather_start(0, b0 + 2)

        gather_wait(1, b0 + 1)
        compute(1, b0 + 1)

        @pl.when(b0 + 3 < ROWS_PER_WORKER)
        def _():
            gather_start(1, b0 + 3)

        return carry

    lax.fori_loop(0, ROWS_PER_WORKER // 2, group, 0)

    pltpu.sync_copy(dbuf, d_hbm.at[pl.ds(wid * ROWS_PER_WORKER,
                                         ROWS_PER_WORKER)])


def _tc_epilogue(d_ref, b_ref, o0_ref, o1_ref):
    delta = jnp.sum(d_ref[...], axis=-1) + (b_ref[1] - b_ref[0])
    sp = jnp.maximum(delta, 0.0) + jnp.log(1.0 + jnp.exp(-jnp.abs(delta)))
    o0_ref[...] = -sp
    o1_ref[...] = delta - sp


def kernel(input, emb_table, fc_w, fc_b):
    idx = input.astype(jnp.int32).reshape(
        NUM_WORKERS, ROWS_PER_WORKER, 2, HALF_L)

    table_pad = jnp.pad(emb_table, ((0, 0), (0, DPAD - EMBED_DIM)))

    wd = (fc_w[:, 1] - fc_w[:, 0]).reshape(MAX_LEN, EMBED_DIM)
    wd_pad = jnp.zeros((MAX_LEN, DPAD), jnp.float32)
    wd_pad = wd_pad.at[:, :EMBED_DIM].set(wd)

    mesh = plsc.VectorSubcoreMesh(core_axis_name="c", subcore_axis_name="s")
    delta16 = pl.kernel(
        _sc_delta_kernel,
        mesh=mesh,
        compiler_params=pltpu.CompilerParams(use_tc_tiling_on_sc=False),
        out_type=jax.ShapeDtypeStruct((BATCH, 16), jnp.float32),
        scratch_types=[
            pltpu.VMEM((ROWS_PER_WORKER, 2, HALF_L), jnp.int32),
            pltpu.VMEM((MAX_LEN, DPAD), jnp.float32),
            pltpu.VMEM((2, 2, HALF_L, DPAD), jnp.float32),
            pltpu.VMEM((ROWS_PER_WORKER, 16), jnp.float32),
            pltpu.SemaphoreType.DMA((2,)),
        ],
    )(idx, wd_pad, table_pad)

    o0, o1 = pl.pallas_call(
        _tc_epilogue,
        out_shape=(
            jax.ShapeDtypeStruct((32, 128), jnp.float32),
            jax.ShapeDtypeStruct((32, 128), jnp.float32),
        ),
        in_specs=(
            pl.BlockSpec(memory_space=pltpu.VMEM),
            pl.BlockSpec(memory_space=pltpu.SMEM),
        ),
    )(delta16.reshape(32, 128, 16), fc_b)

    return jnp.stack([o0.reshape(BATCH), o1.reshape(BATCH)], axis=-1)


# speedup vs baseline: 1.2616x; 1.2616x over previous
import jax
import jax.numpy as jnp
from jax import lax
from jax.experimental import pallas as pl
from jax.experimental.pallas import tpu as pltpu
from jax.experimental.pallas import tpu_sc as plsc

VOCAB = 1000000
EMBED_DIM = 100
MAX_LEN = 200
BATCH = 4096

NUM_WORKERS = 32
ROWS_PER_WORKER = BATCH // NUM_WORKERS
HALF_L = MAX_LEN // 2
DPAD = 128
NCHUNK = DPAD // 16
_NACC = 4


def _sc_delta_kernel(idx_hbm, wd_hbm, table_hbm, d_hbm,
                     idx_v, wd_v, bufs, dbuf, sems):
    wid = lax.axis_index("s") * 2 + lax.axis_index("c")

    pltpu.sync_copy(idx_hbm.at[wid], idx_v)
    pltpu.sync_copy(wd_hbm, wd_v)

    def gather_start(slot, b):
        for j in range(2):
            pltpu.make_async_copy(
                table_hbm.at[idx_v.at[b, j]], bufs.at[slot, j], sems.at[slot]
            ).start()

    def gather_wait(slot, b):
        for j in range(2):
            pltpu.make_async_copy(
                table_hbm.at[idx_v.at[b, j]], bufs.at[slot, j], sems.at[slot]
            ).wait()

    def compute(slot, b):
        accs = [jnp.zeros((16,), jnp.float32) for _ in range(_NACC)]

        def lbody(l, accs):
            accs = list(accs)
            k = 0
            for j in range(2):
                for c in range(NCHUNK):
                    r = bufs[slot, j, l, pl.ds(c * 16, 16)]
                    w = wd_v[j * HALF_L + l, pl.ds(c * 16, 16)]
                    accs[k % _NACC] = accs[k % _NACC] + r * w
                    k += 1
            return tuple(accs)

        accs = lax.fori_loop(0, HALF_L, lbody, tuple(accs), unroll=2)
        dbuf[b] = accs[0] + accs[1] + accs[2] + accs[3]

    gather_start(0, 0)
    gather_start(1, 1)

    def group(g, carry):
        b0 = 2 * g
        gather_wait(0, b0)
        compute(0, b0)

        @pl.when(b0 + 2 < ROWS_PER_WORKER)
        def _():
            gather_start(0, b0 + 2)

        gather_wait(1, b0 + 1)
        compute(1, b0 + 1)

        @pl.when(b0 + 3 < ROWS_PER_WORKER)
        def _():
            gather_start(1, b0 + 3)

        return carry

    lax.fori_loop(0, ROWS_PER_WORKER // 2, group, 0)

    pltpu.sync_copy(dbuf, d_hbm.at[pl.ds(wid * ROWS_PER_WORKER,
                                         ROWS_PER_WORKER)])


def _tc_epilogue(d_ref, b_ref, o0_ref, o1_ref):
    delta = jnp.sum(d_ref[...], axis=-1) + (b_ref[1] - b_ref[0])
    sp = jnp.maximum(delta, 0.0) + jnp.log(1.0 + jnp.exp(-jnp.abs(delta)))
    o0_ref[...] = -sp
    o1_ref[...] = delta - sp


def kernel(input, emb_table, fc_w, fc_b):
    idx = input.astype(jnp.int32).reshape(
        NUM_WORKERS, ROWS_PER_WORKER, 2, HALF_L)

    table_pad = jnp.pad(emb_table, ((0, 0), (0, DPAD - EMBED_DIM)))

    wd = (fc_w[:, 1] - fc_w[:, 0]).reshape(MAX_LEN, EMBED_DIM)
    wd_pad = jnp.zeros((MAX_LEN, DPAD), jnp.float32)
    wd_pad = wd_pad.at[:, :EMBED_DIM].set(wd)

    mesh = plsc.VectorSubcoreMesh(core_axis_name="c", subcore_axis_name="s")
    delta16 = pl.kernel(
        _sc_delta_kernel,
        mesh=mesh,
        out_type=jax.ShapeDtypeStruct((BATCH, 16), jnp.float32),
        scratch_types=[
            pltpu.VMEM((ROWS_PER_WORKER, 2, HALF_L), jnp.int32),
            pltpu.VMEM((MAX_LEN, DPAD), jnp.float32),
            pltpu.VMEM((2, 2, HALF_L, DPAD), jnp.float32),
            pltpu.VMEM((ROWS_PER_WORKER, 16), jnp.float32),
            pltpu.SemaphoreType.DMA((2,)),
        ],
    )(idx, wd_pad, table_pad)

    o0, o1 = pl.pallas_call(
        _tc_epilogue,
        out_shape=(
            jax.ShapeDtypeStruct((32, 128), jnp.float32),
            jax.ShapeDtypeStruct((32, 128), jnp.float32),
        ),
        in_specs=(
            pl.BlockSpec(memory_space=pltpu.VMEM),
            pl.BlockSpec(memory_space=pltpu.SMEM),
        ),
    )(delta16.reshape(32, 128, 16), fc_b)

    return jnp.stack([o0.reshape(BATCH), o1.reshape(BATCH)], axis=-1)


# speedup vs baseline: 2.7631x; 2.1903x over previous
import jax
import jax.numpy as jnp
from jax import lax
from jax.experimental import pallas as pl
from jax.experimental.pallas import tpu as pltpu
from jax.experimental.pallas import tpu_sc as plsc

VOCAB = 1000000
EMBED_DIM = 100
MAX_LEN = 200
BATCH = 4096

NUM_WORKERS = 32
ROWS_PER_WORKER = BATCH // NUM_WORKERS
HALF_L = MAX_LEN // 2
DPAD = 128
NCHUNK = DPAD // 16
_NACC = 4


def _sc_delta_kernel(idx_hbm, wd_hbm, table_hbm, d_hbm,
                     idx_v, wd_v, bufs, dbuf, sems):
    wid = lax.axis_index("s") * 2 + lax.axis_index("c")

    pltpu.sync_copy(idx_hbm.at[wid], idx_v)
    pltpu.sync_copy(wd_hbm, wd_v)

    def gather_start(slot, b):
        for j in range(2):
            pltpu.make_async_copy(
                table_hbm.at[idx_v.at[b, j]], bufs.at[slot, j], sems.at[slot]
            ).start()

    def gather_wait(slot, b):
        for j in range(2):
            pltpu.make_async_copy(
                table_hbm.at[idx_v.at[b, j]], bufs.at[slot, j], sems.at[slot]
            ).wait()

    def compute(slot, b):
        accs = [jnp.zeros((16,), jnp.float32) for _ in range(_NACC)]

        def lbody(l, accs):
            accs = list(accs)
            k = 0
            for j in range(2):
                for c in range(NCHUNK):
                    r = bufs[slot, j, l, pl.ds(c * 16, 16)]
                    w = wd_v[j * HALF_L + l, pl.ds(c * 16, 16)]
                    accs[k % _NACC] = accs[k % _NACC] + r * w
                    k += 1
            return tuple(accs)

        accs = lax.fori_loop(0, HALF_L, lbody, tuple(accs), unroll=2)
        dbuf[b] = accs[0] + accs[1] + accs[2] + accs[3]

    gather_start(0, 0)
    gather_start(1, 1)

    def group(g, carry):
        b0 = 2 * g
        gather_wait(0, b0)
        compute(0, b0)

        @pl.when(b0 + 2 < ROWS_PER_WORKER)
        def _():
            gather_start(0, b0 + 2)

        gather_wait(1, b0 + 1)
        compute(1, b0 + 1)

        @pl.when(b0 + 3 < ROWS_PER_WORKER)
        def _():
            gather_start(1, b0 + 3)

        return carry

    lax.fori_loop(0, ROWS_PER_WORKER // 2, group, 0)

    pltpu.sync_copy(dbuf, d_hbm.at[pl.ds(wid * ROWS_PER_WORKER,
                                         ROWS_PER_WORKER)])


_PAD_BLK = 8192


def _tc_pad_kernel(src_ref, dst_ref):
    dst_ref[...] = jnp.concatenate(
        [src_ref[...],
         jnp.zeros((_PAD_BLK, DPAD - EMBED_DIM), jnp.float32)], axis=1)


def _tc_epilogue(d_ref, b_ref, o0_ref, o1_ref):
    delta = jnp.sum(d_ref[...], axis=-1) + (b_ref[1] - b_ref[0])
    sp = jnp.maximum(delta, 0.0) + jnp.log(1.0 + jnp.exp(-jnp.abs(delta)))
    o0_ref[...] = -sp
    o1_ref[...] = delta - sp


def kernel(input, emb_table, fc_w, fc_b):
    idx = input.astype(jnp.int32).reshape(
        NUM_WORKERS, ROWS_PER_WORKER, 2, HALF_L)

    table_pad = pl.pallas_call(
        _tc_pad_kernel,
        grid=(VOCAB // _PAD_BLK,),
        in_specs=(pl.BlockSpec((_PAD_BLK, EMBED_DIM), lambda i: (i, 0)),),
        out_specs=pl.BlockSpec((_PAD_BLK, DPAD), lambda i: (i, 0)),
        out_shape=jax.ShapeDtypeStruct((VOCAB, DPAD), jnp.float32),
    )(emb_table)

    wd = (fc_w[:, 1] - fc_w[:, 0]).reshape(MAX_LEN, EMBED_DIM)
    wd_pad = jnp.zeros((MAX_LEN, DPAD), jnp.float32)
    wd_pad = wd_pad.at[:, :EMBED_DIM].set(wd)

    mesh = plsc.VectorSubcoreMesh(core_axis_name="c", subcore_axis_name="s")
    delta16 = pl.kernel(
        _sc_delta_kernel,
        mesh=mesh,
        out_type=jax.ShapeDtypeStruct((BATCH, 16), jnp.float32),
        scratch_types=[
            pltpu.VMEM((ROWS_PER_WORKER, 2, HALF_L), jnp.int32),
            pltpu.VMEM((MAX_LEN, DPAD), jnp.float32),
            pltpu.VMEM((2, 2, HALF_L, DPAD), jnp.float32),
            pltpu.VMEM((ROWS_PER_WORKER, 16), jnp.float32),
            pltpu.SemaphoreType.DMA((2,)),
        ],
    )(idx, wd_pad, table_pad)

    o0, o1 = pl.pallas_call(
        _tc_epilogue,
        out_shape=(
            jax.ShapeDtypeStruct((32, 128), jnp.float32),
            jax.ShapeDtypeStruct((32, 128), jnp.float32),
        ),
        in_specs=(
            pl.BlockSpec(memory_space=pltpu.VMEM),
            pl.BlockSpec(memory_space=pltpu.SMEM),
        ),
    )(delta16.reshape(32, 128, 16), fc_b)

    return jnp.stack([o0.reshape(BATCH), o1.reshape(BATCH)], axis=-1)


# speedup vs baseline: 2.8171x; 1.0195x over previous
import jax
import jax.numpy as jnp
from jax import lax
from jax.experimental import pallas as pl
from jax.experimental.pallas import tpu as pltpu
from jax.experimental.pallas import tpu_sc as plsc

VOCAB = 1000000
EMBED_DIM = 100
MAX_LEN = 200
BATCH = 4096

NUM_WORKERS = 32
ROWS_PER_WORKER = BATCH // NUM_WORKERS
HALF_L = MAX_LEN // 2
DPAD = 128
NCHUNK = 7
_NACC = 4


def _sc_delta_kernel(idx_hbm, wd_hbm, table_hbm, d_hbm,
                     idx_v, wd_v, bufs, dbuf, sems):
    wid = lax.axis_index("s") * 2 + lax.axis_index("c")

    pltpu.sync_copy(idx_hbm.at[wid], idx_v)
    pltpu.sync_copy(wd_hbm, wd_v)

    def gather_start(slot, b):
        for j in range(2):
            pltpu.make_async_copy(
                table_hbm.at[idx_v.at[b, j]], bufs.at[slot, j], sems.at[slot]
            ).start()

    def gather_wait(slot, b):
        for j in range(2):
            pltpu.make_async_copy(
                table_hbm.at[idx_v.at[b, j]], bufs.at[slot, j], sems.at[slot]
            ).wait()

    def compute(slot, b):
        accs = [jnp.zeros((16,), jnp.float32) for _ in range(_NACC)]

        def lbody(l, accs):
            accs = list(accs)
            k = 0
            for j in range(2):
                for c in range(NCHUNK):
                    r = bufs[slot, j, l, pl.ds(c * 16, 16)]
                    w = wd_v[j * HALF_L + l, pl.ds(c * 16, 16)]
                    accs[k % _NACC] = accs[k % _NACC] + r * w
                    k += 1
            return tuple(accs)

        accs = lax.fori_loop(0, HALF_L, lbody, tuple(accs), unroll=2)
        dbuf[b] = accs[0] + accs[1] + accs[2] + accs[3]

    gather_start(0, 0)
    gather_start(1, 1)

    def group(g, carry):
        b0 = 2 * g
        gather_wait(0, b0)
        compute(0, b0)

        @pl.when(b0 + 2 < ROWS_PER_WORKER)
        def _():
            gather_start(0, b0 + 2)

        gather_wait(1, b0 + 1)
        compute(1, b0 + 1)

        @pl.when(b0 + 3 < ROWS_PER_WORKER)
        def _():
            gather_start(1, b0 + 3)

        return carry

    lax.fori_loop(0, ROWS_PER_WORKER // 2, group, 0)

    pltpu.sync_copy(dbuf, d_hbm.at[pl.ds(wid * ROWS_PER_WORKER,
                                         ROWS_PER_WORKER)])


_PAD_BLK = 20000


def _tc_pad_kernel(src_ref, dst_ref):
    dst_ref[:, :112] = jnp.concatenate(
        [src_ref[...], jnp.zeros((_PAD_BLK, 12), jnp.float32)], axis=1)


def _tc_epilogue(d_ref, b_ref, o0_ref, o1_ref):
    delta = jnp.sum(d_ref[...], axis=-1) + (b_ref[1] - b_ref[0])
    sp = jnp.maximum(delta, 0.0) + jnp.log(1.0 + jnp.exp(-jnp.abs(delta)))
    o0_ref[...] = -sp
    o1_ref[...] = delta - sp


def kernel(input, emb_table, fc_w, fc_b):
    idx = input.astype(jnp.int32).reshape(
        NUM_WORKERS, ROWS_PER_WORKER, 2, HALF_L)

    table_pad = pl.pallas_call(
        _tc_pad_kernel,
        grid=(VOCAB // _PAD_BLK,),
        in_specs=(pl.BlockSpec((_PAD_BLK, EMBED_DIM), lambda i: (i, 0)),),
        out_specs=pl.BlockSpec((_PAD_BLK, DPAD), lambda i: (i, 0)),
        out_shape=jax.ShapeDtypeStruct((VOCAB, DPAD), jnp.float32),
    )(emb_table)

    wd = (fc_w[:, 1] - fc_w[:, 0]).reshape(MAX_LEN, EMBED_DIM)
    wd_pad = jnp.zeros((MAX_LEN, DPAD), jnp.float32)
    wd_pad = wd_pad.at[:, :EMBED_DIM].set(wd)

    mesh = plsc.VectorSubcoreMesh(core_axis_name="c", subcore_axis_name="s")
    delta16 = pl.kernel(
        _sc_delta_kernel,
        mesh=mesh,
        out_type=jax.ShapeDtypeStruct((BATCH, 16), jnp.float32),
        scratch_types=[
            pltpu.VMEM((ROWS_PER_WORKER, 2, HALF_L), jnp.int32),
            pltpu.VMEM((MAX_LEN, DPAD), jnp.float32),
            pltpu.VMEM((2, 2, HALF_L, DPAD), jnp.float32),
            pltpu.VMEM((ROWS_PER_WORKER, 16), jnp.float32),
            pltpu.SemaphoreType.DMA((2,)),
        ],
    )(idx, wd_pad, table_pad)

    o0, o1 = pl.pallas_call(
        _tc_epilogue,
        out_shape=(
            jax.ShapeDtypeStruct((32, 128), jnp.float32),
            jax.ShapeDtypeStruct((32, 128), jnp.float32),
        ),
        in_specs=(
            pl.BlockSpec(memory_space=pltpu.VMEM),
            pl.BlockSpec(memory_space=pltpu.SMEM),
        ),
    )(delta16.reshape(32, 128, 16), fc_b)

    return jnp.stack([o0.reshape(BATCH), o1.reshape(BATCH)], axis=-1)


# speedup vs baseline: 2.8181x; 1.0004x over previous
import jax
import jax.numpy as jnp
from jax import lax
from jax.experimental import pallas as pl
from jax.experimental.pallas import tpu as pltpu
from jax.experimental.pallas import tpu_sc as plsc

VOCAB = 1000000
EMBED_DIM = 100
MAX_LEN = 200
BATCH = 4096

NUM_WORKERS = 32
ROWS_PER_WORKER = BATCH // NUM_WORKERS
HALF_L = MAX_LEN // 2
DPAD = 128
NCHUNK = 7
_NACC = 4


def _sc_delta_kernel(idx_hbm, wd_hbm, table_hbm, d_hbm,
                     idx_v, wd_v, bufs, dbuf, sems):
    wid = lax.axis_index("s") * 2 + lax.axis_index("c")

    pltpu.sync_copy(idx_hbm.at[wid], idx_v)
    pltpu.sync_copy(wd_hbm, wd_v)

    def gather_start(slot, b):
        for j in range(2):
            pltpu.make_async_copy(
                table_hbm.at[idx_v.at[b, j]], bufs.at[slot, j], sems.at[slot]
            ).start()

    def gather_wait(slot, b):
        for j in range(2):
            pltpu.make_async_copy(
                table_hbm.at[idx_v.at[b, j]], bufs.at[slot, j], sems.at[slot]
            ).wait()

    def compute(slot, b):
        accs = [jnp.zeros((16,), jnp.float32) for _ in range(_NACC)]

        def lbody(l, accs):
            accs = list(accs)
            k = 0
            for j in range(2):
                for c in range(NCHUNK):
                    r = bufs[slot, j, l, pl.ds(c * 16, 16)]
                    w = wd_v[j * HALF_L + l, pl.ds(c * 16, 16)]
                    accs[k % _NACC] = accs[k % _NACC] + r * w
                    k += 1
            return tuple(accs)

        accs = lax.fori_loop(0, HALF_L, lbody, tuple(accs), unroll=2)
        dbuf[b] = accs[0] + accs[1] + accs[2] + accs[3]

    gather_start(0, 0)
    gather_start(1, 1)

    def group(g, carry):
        b0 = 2 * g
        gather_wait(0, b0)
        compute(0, b0)

        @pl.when(b0 + 2 < ROWS_PER_WORKER)
        def _():
            gather_start(0, b0 + 2)

        gather_wait(1, b0 + 1)
        compute(1, b0 + 1)

        @pl.when(b0 + 3 < ROWS_PER_WORKER)
        def _():
            gather_start(1, b0 + 3)

        return carry

    lax.fori_loop(0, ROWS_PER_WORKER // 2, group, 0)

    pltpu.sync_copy(dbuf, d_hbm.at[pl.ds(wid * ROWS_PER_WORKER,
                                         ROWS_PER_WORKER)])


_PAD_BLK = 25000


def _tc_pad_kernel(src_ref, dst_ref):
    dst_ref[:, :112] = jnp.concatenate(
        [src_ref[...], jnp.zeros((_PAD_BLK, 12), jnp.float32)], axis=1)


def _tc_epilogue(d_ref, b_ref, o0_ref, o1_ref):
    delta = jnp.sum(d_ref[...], axis=-1) + (b_ref[1] - b_ref[0])
    sp = jnp.maximum(delta, 0.0) + jnp.log(1.0 + jnp.exp(-jnp.abs(delta)))
    o0_ref[...] = -sp
    o1_ref[...] = delta - sp


def kernel(input, emb_table, fc_w, fc_b):
    idx = input.astype(jnp.int32).reshape(
        NUM_WORKERS, ROWS_PER_WORKER, 2, HALF_L)

    table_pad = pl.pallas_call(
        _tc_pad_kernel,
        grid=(VOCAB // _PAD_BLK,),
        in_specs=(pl.BlockSpec((_PAD_BLK, EMBED_DIM), lambda i: (i, 0)),),
        out_specs=pl.BlockSpec((_PAD_BLK, DPAD), lambda i: (i, 0)),
        out_shape=jax.ShapeDtypeStruct((VOCAB, DPAD), jnp.float32),
    )(emb_table)

    wd = (fc_w[:, 1] - fc_w[:, 0]).reshape(MAX_LEN, EMBED_DIM)
    wd_pad = jnp.zeros((MAX_LEN, DPAD), jnp.float32)
    wd_pad = wd_pad.at[:, :EMBED_DIM].set(wd)

    mesh = plsc.VectorSubcoreMesh(core_axis_name="c", subcore_axis_name="s")
    delta16 = pl.kernel(
        _sc_delta_kernel,
        mesh=mesh,
        out_type=jax.ShapeDtypeStruct((BATCH, 16), jnp.float32),
        scratch_types=[
            pltpu.VMEM((ROWS_PER_WORKER, 2, HALF_L), jnp.int32),
            pltpu.VMEM((MAX_LEN, DPAD), jnp.float32),
            pltpu.VMEM((2, 2, HALF_L, DPAD), jnp.float32),
            pltpu.VMEM((ROWS_PER_WORKER, 16), jnp.float32),
            pltpu.SemaphoreType.DMA((2,)),
        ],
    )(idx, wd_pad, table_pad)

    o0, o1 = pl.pallas_call(
        _tc_epilogue,
        out_shape=(
            jax.ShapeDtypeStruct((32, 128), jnp.float32),
            jax.ShapeDtypeStruct((32, 128), jnp.float32),
        ),
        in_specs=(
            pl.BlockSpec(memory_space=pltpu.VMEM),
            pl.BlockSpec(memory_space=pltpu.SMEM),
        ),
    )(delta16.reshape(32, 128, 16), fc_b)

    return jnp.stack([o0.reshape(BATCH), o1.reshape(BATCH)], axis=-1)
